# Initial kernel scaffold; baseline (speedup 1.0000x reference)
import jax
import jax.numpy as jnp
from jax.experimental import pallas as pl


def kernel(pred_boxes, pred_scores, mask_scores, pred_masks, gt_boxes):
    raise NotImplementedError("write your pallas kernel here")


# speedup vs baseline: 374.1848x; 374.1848x over previous
import jax
import jax.numpy as jnp
from jax.experimental import pallas as pl
from jax.experimental.pallas import tpu as pltpu

_IOU_THR = 0.5
_SCORE_THR = 0.05
_M = 20000
_MP = 20480
_N = 100
_HW = 784
_CL = 1024
_BIG = 2 ** 30


def _boxseg_kernel(pbt_ref, gb_ref, ps_ref, ms_ref, pm_ref,
                   gm_ref, fl_ref, os_ref, dtm_ref, biou_ref,
                   iout_ref, ws_ref, sm_ref, gtma_ref, gtms_ref, sems):
    nchunk = _MP // _CL

    dtm_ref[:, :] = jnp.zeros((1, _MP), jnp.int32)
    fl_ref[:, :] = jnp.zeros((_N, 1), jnp.float32)
    os_ref[:, :] = jnp.zeros((_N, 1), jnp.float32)
    biou_ref[:, :] = jnp.zeros((_N, 1), jnp.float32)
    gtma_ref[:, :] = jnp.full((_N, 1), -1, jnp.int32)
    for j in range(_N):
        gtms_ref[0, j] = jnp.int32(-1)
    ps = ps_ref[:, :]
    ws_ref[:, :] = jnp.where(ps >= _SCORE_THR, ps, -1.0)

    gx1 = gb_ref[:, 0:1]
    gy1 = gb_ref[:, 1:2]
    gx2 = gb_ref[:, 2:3]
    gy2 = gb_ref[:, 3:4]
    garea = (gx2 - gx1) * (gy2 - gy1)
    for c in range(nchunk):
        sl = pl.ds(c * _CL, _CL)
        px1 = pbt_ref[0:1, sl]
        py1 = pbt_ref[1:2, sl]
        px2 = pbt_ref[2:3, sl]
        py2 = pbt_ref[3:4, sl]
        parea = (px2 - px1) * (py2 - py1)
        w = jnp.maximum(jnp.minimum(px2, gx2) - jnp.maximum(px1, gx1), 0.0)
        h = jnp.maximum(jnp.minimum(py2, gy2) - jnp.maximum(py1, gy1), 0.0)
        inter = w * h
        union = jnp.maximum(parea + garea - inter, 1e-6)
        iout_ref[:, sl] = inter / union

    def event_body(t, carry):
        availa = gtma_ref[:, :] <= 0
        vmax = -2.0
        for c in range(nchunk):
            sl = pl.ds(c * _CL, _CL)
            cand = jnp.where(availa, iout_ref[:, sl], -1.0)
            cm = jnp.max(cand, axis=0, keepdims=True)
            smc = jnp.where(cm >= _IOU_THR, ws_ref[0:1, sl], -1.0)
            sm_ref[0:1, sl] = smc
            vmax = jnp.maximum(vmax, jnp.max(smc))
        ok = vmax >= _SCORE_THR

        @pl.when(ok)
        def _():
            istar = _BIG
            msval = 0.0
            for c in range(nchunk):
                sl = pl.ds(c * _CL, _CL)
                ids = jax.lax.broadcasted_iota(jnp.int32, (1, _CL), 1) + c * _CL
                smc = sm_ref[0:1, sl]
                istar = jnp.minimum(
                    istar, jnp.min(jnp.where(smc == vmax, ids, _BIG)))
            rowv = jnp.full((_N, 1), -1.0, jnp.float32)
            for c in range(nchunk):
                sl = pl.ds(c * _CL, _CL)
                ids = jax.lax.broadcasted_iota(jnp.int32, (1, _CL), 1) + c * _CL
                hit = ids == istar
                seg = jnp.where(hit, iout_ref[:, sl], -1.0)
                rowv = jnp.maximum(rowv, jnp.max(seg, axis=1, keepdims=True))
                msval = msval + jnp.sum(jnp.where(hit, ms_ref[0:1, sl], 0.0))
            candrow = jnp.where(availa, rowv, -1.0)
            vrow = jnp.max(candrow)
            gida = jax.lax.broadcasted_iota(jnp.int32, (_N, 1), 0)
            mstar = jnp.min(jnp.where(candrow == vrow, gida, _BIG))
            for c in range(nchunk):
                sl = pl.ds(c * _CL, _CL)
                ids = jax.lax.broadcasted_iota(jnp.int32, (1, _CL), 1) + c * _CL
                hit = ids == istar
                dtm_ref[0:1, sl] = jnp.where(hit, mstar, dtm_ref[0:1, sl])
                ws_ref[0:1, sl] = jnp.where(hit, -1.0, ws_ref[0:1, sl])
            hita = gida == mstar
            gtma_ref[:, :] = jnp.where(hita, istar, gtma_ref[:, :])
            gtms_ref[0, mstar] = istar
            biou_ref[:, :] = jnp.where(hita, vrow, biou_ref[:, :])
            os_ref[:, :] = jnp.where(hita, vmax, os_ref[:, :])
            fl_ref[:, :] = jnp.where(hita, msval, fl_ref[:, :])
        return 0

    jax.lax.fori_loop(0, _N + 1, event_body, 0)

    copies = []
    for j in range(_N):
        src = jnp.maximum(gtms_ref[0, j], 0)
        cp = pltpu.make_async_copy(
            pm_ref.at[pl.ds(src, 1), :], gm_ref.at[pl.ds(j, 1), :], sems.at[j])
        cp.start()
        copies.append(cp)
    for cp in copies:
        cp.wait()
    for j in range(_N):
        matched = (gtms_ref[0, j] > -1).astype(jnp.float32)
        gm_ref[pl.ds(j, 1), :] = gm_ref[pl.ds(j, 1), :] * matched


@jax.jit
def kernel(pred_boxes, pred_scores, mask_scores, pred_masks, gt_boxes):
    pm2 = pred_masks.reshape(_M, _HW)
    pad = _MP - _M
    pbt = jnp.pad(pred_boxes, ((0, pad), (0, 0))).T
    ps2 = jnp.pad(pred_scores, (0, pad), constant_values=-1.0).reshape(1, _MP)
    ms2 = jnp.pad(mask_scores, (0, pad)).reshape(1, _MP)
    out_shape = (
        jax.ShapeDtypeStruct((_N, _HW), jnp.float32),
        jax.ShapeDtypeStruct((_N, 1), jnp.float32),
        jax.ShapeDtypeStruct((_N, 1), jnp.float32),
        jax.ShapeDtypeStruct((1, _MP), jnp.int32),
        jax.ShapeDtypeStruct((_N, 1), jnp.float32),
    )
    vmem = pl.BlockSpec(memory_space=pltpu.MemorySpace.VMEM)
    gm, fl, osc, dtm, biou = pl.pallas_call(
        _boxseg_kernel,
        out_shape=out_shape,
        in_specs=[vmem, vmem, vmem, vmem,
                  pl.BlockSpec(memory_space=pltpu.MemorySpace.HBM)],
        out_specs=(vmem, vmem, vmem, vmem, vmem),
        scratch_shapes=[
            pltpu.VMEM((_N, _MP), jnp.float32),
            pltpu.VMEM((1, _MP), jnp.float32),
            pltpu.VMEM((1, _MP), jnp.float32),
            pltpu.VMEM((_N, 1), jnp.int32),
            pltpu.SMEM((1, _N), jnp.int32),
            pltpu.SemaphoreType.DMA((_N,)),
        ],
    )(pbt, gt_boxes, ps2, ms2, pm2)
    return (gm.reshape(_N, 28, 28), fl.reshape(_N), osc.reshape(_N),
            dtm[0, :_M], biou.reshape(_N))


# speedup vs baseline: 463.8548x; 1.2396x over previous
import jax
import jax.numpy as jnp
from jax.experimental import pallas as pl
from jax.experimental.pallas import tpu as pltpu

_IOU_THR = 0.5
_SCORE_THR = 0.05
_M = 20000
_MP = 20480
_N = 100
_HW = 784
_CL = 1024
_BIG = 2 ** 30


def _boxseg_kernel(pbt_ref, gb_ref, ps_ref, ms_ref, pm_ref,
                   gm_ref, fl_ref, os_ref, dtm_ref, biou_ref,
                   iout_ref, ws_ref, sm_ref, gtma_ref, gtms_ref, sems):
    nchunk = _MP // _CL

    dtm_ref[:, :] = jnp.zeros((1, _MP), jnp.int32)
    fl_ref[:, :] = jnp.zeros((_N, 1), jnp.float32)
    os_ref[:, :] = jnp.zeros((_N, 1), jnp.float32)
    biou_ref[:, :] = jnp.zeros((_N, 1), jnp.float32)
    gtma_ref[:, :] = jnp.full((_N, 1), -1, jnp.int32)
    for j in range(_N):
        gtms_ref[0, j] = jnp.int32(-1)
    ps = ps_ref[:, :]
    ws_ref[:, :] = jnp.where(ps >= _SCORE_THR, ps, -1.0)

    gx1 = gb_ref[:, 0:1]
    gy1 = gb_ref[:, 1:2]
    gx2 = gb_ref[:, 2:3]
    gy2 = gb_ref[:, 3:4]
    garea = (gx2 - gx1) * (gy2 - gy1)
    for c in range(nchunk):
        sl = pl.ds(c * _CL, _CL)
        px1 = pbt_ref[0:1, sl]
        py1 = pbt_ref[1:2, sl]
        px2 = pbt_ref[2:3, sl]
        py2 = pbt_ref[3:4, sl]
        parea = (px2 - px1) * (py2 - py1)
        w = jnp.maximum(jnp.minimum(px2, gx2) - jnp.maximum(px1, gx1), 0.0)
        h = jnp.maximum(jnp.minimum(py2, gy2) - jnp.maximum(py1, gy1), 0.0)
        inter = w * h
        union = jnp.maximum(parea + garea - inter, 1e-6)
        iout_ref[:, sl] = inter / union

    def event_body(t, carry):
        availa = gtma_ref[:, :] <= 0
        vmax = -2.0
        for c in range(nchunk):
            sl = pl.ds(c * _CL, _CL)
            cand = jnp.where(availa, iout_ref[:, sl], -1.0)
            cm = jnp.max(cand, axis=0, keepdims=True)
            smc = jnp.where(cm >= _IOU_THR, ws_ref[0:1, sl], -1.0)
            sm_ref[0:1, sl] = smc
            vmax = jnp.maximum(vmax, jnp.max(smc))
        ok = vmax >= _SCORE_THR

        @pl.when(ok)
        def _():
            istar = _BIG
            for c in range(nchunk):
                sl = pl.ds(c * _CL, _CL)
                ids = jax.lax.broadcasted_iota(jnp.int32, (1, _CL), 1) + c * _CL
                smc = sm_ref[0:1, sl]
                istar = jnp.minimum(
                    istar, jnp.min(jnp.where(smc == vmax, ids, _BIG)))
            blk = pl.multiple_of((istar // 128) * 128, 128)
            loff = istar - blk
            lids = jax.lax.broadcasted_iota(jnp.int32, (1, 128), 1)
            hitl = lids == loff
            ioblk = iout_ref[:, pl.ds(blk, 128)]
            colv = jnp.max(jnp.where(hitl, ioblk, -1.0), axis=1, keepdims=True)
            msval = jnp.sum(jnp.where(hitl, ms_ref[0:1, pl.ds(blk, 128)], 0.0))
            candrow = jnp.where(availa, colv, -1.0)
            vrow = jnp.max(candrow)
            gida = jax.lax.broadcasted_iota(jnp.int32, (_N, 1), 0)
            mstar = jnp.min(jnp.where(candrow == vrow, gida, _BIG))
            dtm_ref[0:1, pl.ds(blk, 128)] = jnp.where(
                hitl, mstar, dtm_ref[0:1, pl.ds(blk, 128)])
            ws_ref[0:1, pl.ds(blk, 128)] = jnp.where(
                hitl, -1.0, ws_ref[0:1, pl.ds(blk, 128)])
            hita = gida == mstar
            gtma_ref[:, :] = jnp.where(hita, istar, gtma_ref[:, :])
            gtms_ref[0, mstar] = istar
            biou_ref[:, :] = jnp.where(hita, vrow, biou_ref[:, :])
            os_ref[:, :] = jnp.where(hita, vmax, os_ref[:, :])
            fl_ref[:, :] = jnp.where(hita, msval, fl_ref[:, :])
        return 0

    jax.lax.fori_loop(0, _N + 1, event_body, 0)

    copies = []
    for j in range(_N):
        src = jnp.maximum(gtms_ref[0, j], 0)
        cp = pltpu.make_async_copy(
            pm_ref.at[pl.ds(src, 1), :], gm_ref.at[pl.ds(j, 1), :], sems.at[j])
        cp.start()
        copies.append(cp)
    for cp in copies:
        cp.wait()
    for j in range(_N):
        matched = (gtms_ref[0, j] > -1).astype(jnp.float32)
        gm_ref[pl.ds(j, 1), :] = gm_ref[pl.ds(j, 1), :] * matched


@jax.jit
def kernel(pred_boxes, pred_scores, mask_scores, pred_masks, gt_boxes):
    pm2 = pred_masks.reshape(_M, _HW)
    pad = _MP - _M
    pbt = jnp.pad(pred_boxes, ((0, pad), (0, 0))).T
    ps2 = jnp.pad(pred_scores, (0, pad), constant_values=-1.0).reshape(1, _MP)
    ms2 = jnp.pad(mask_scores, (0, pad)).reshape(1, _MP)
    out_shape = (
        jax.ShapeDtypeStruct((_N, _HW), jnp.float32),
        jax.ShapeDtypeStruct((_N, 1), jnp.float32),
        jax.ShapeDtypeStruct((_N, 1), jnp.float32),
        jax.ShapeDtypeStruct((1, _MP), jnp.int32),
        jax.ShapeDtypeStruct((_N, 1), jnp.float32),
    )
    vmem = pl.BlockSpec(memory_space=pltpu.MemorySpace.VMEM)
    gm, fl, osc, dtm, biou = pl.pallas_call(
        _boxseg_kernel,
        out_shape=out_shape,
        in_specs=[vmem, vmem, vmem, vmem,
                  pl.BlockSpec(memory_space=pltpu.MemorySpace.HBM)],
        out_specs=(vmem, vmem, vmem, vmem, vmem),
        scratch_shapes=[
            pltpu.VMEM((_N, _MP), jnp.float32),
            pltpu.VMEM((1, _MP), jnp.float32),
            pltpu.VMEM((1, _MP), jnp.float32),
            pltpu.VMEM((_N, 1), jnp.int32),
            pltpu.SMEM((1, _N), jnp.int32),
            pltpu.SemaphoreType.DMA((_N,)),
        ],
    )(pbt, gt_boxes, ps2, ms2, pm2)
    return (gm.reshape(_N, 28, 28), fl.reshape(_N), osc.reshape(_N),
            dtm[0, :_M], biou.reshape(_N))


# speedup vs baseline: 618.4802x; 1.3333x over previous
import jax
import jax.numpy as jnp
from jax.experimental import pallas as pl
from jax.experimental.pallas import tpu as pltpu

_IOU_THR = 0.5
_SCORE_THR = 0.05
_M = 20000
_MP = 20480
_N = 100
_HW = 784
_CL = 1024
_BIG = 2 ** 30


def _boxseg_kernel(pbt_ref, gb_ref, ps_ref, ms_ref, pm_ref,
                   gm_ref, fl_ref, os_ref, dtm_ref, biou_ref,
                   iout_ref, ws_ref, sm_ref, gtma_ref, gtms_ref,
                   wb_ref, aw_ref, sems):
    nchunk = _MP // _CL

    dtm_ref[:, :] = jnp.zeros((1, _MP), jnp.int32)
    fl_ref[:, :] = jnp.zeros((_N, 1), jnp.float32)
    os_ref[:, :] = jnp.zeros((_N, 1), jnp.float32)
    biou_ref[:, :] = jnp.zeros((_N, 1), jnp.float32)
    gtma_ref[:, :] = jnp.full((_N, 1), -1, jnp.int32)
    for j in range(_N):
        gtms_ref[0, j] = jnp.int32(-1)
    ps = ps_ref[:, :]
    ws_ref[:, :] = jnp.where(ps >= _SCORE_THR, ps, -1.0)

    gx1 = gb_ref[:, 0:1]
    gy1 = gb_ref[:, 1:2]
    gx2 = gb_ref[:, 2:3]
    gy2 = gb_ref[:, 3:4]
    garea = (gx2 - gx1) * (gy2 - gy1)
    for c in range(nchunk):
        sl = pl.ds(c * _CL, _CL)
        px1 = pbt_ref[0:1, sl]
        py1 = pbt_ref[1:2, sl]
        px2 = pbt_ref[2:3, sl]
        py2 = pbt_ref[3:4, sl]
        parea = (px2 - px1) * (py2 - py1)
        w = jnp.maximum(jnp.minimum(px2, gx2) - jnp.maximum(px1, gx1), 0.0)
        h = jnp.maximum(jnp.minimum(py2, gy2) - jnp.maximum(py1, gy1), 0.0)
        inter = w * h
        union = jnp.maximum(parea + garea - inter, 1e-6)
        iout_ref[:, sl] = inter / union

    gida0 = jax.lax.broadcasted_iota(jnp.int32, (_N, 1), 0)
    wts = jnp.left_shift(1, gida0 % 32)
    for c in range(nchunk):
        sl = pl.ds(c * _CL, _CL)
        wvals = jnp.where(iout_ref[:, sl] >= _IOU_THR, wts, 0)
        wb_ref[0:1, sl] = jnp.sum(wvals[0:32, :], axis=0, keepdims=True)
        wb_ref[1:2, sl] = jnp.sum(wvals[32:64, :], axis=0, keepdims=True)
        wb_ref[2:3, sl] = jnp.sum(wvals[64:96, :], axis=0, keepdims=True)
        wb_ref[3:4, sl] = jnp.sum(wvals[96:100, :], axis=0, keepdims=True)
    aw_ref[0, 0] = jnp.int32(-1)
    aw_ref[0, 1] = jnp.int32(-1)
    aw_ref[0, 2] = jnp.int32(-1)
    aw_ref[0, 3] = jnp.int32(15)

    def event_body(t, carry):
        a0 = aw_ref[0, 0]
        a1 = aw_ref[0, 1]
        a2 = aw_ref[0, 2]
        a3 = aw_ref[0, 3]
        vmax = -2.0
        for c in range(nchunk):
            sl = pl.ds(c * _CL, _CL)
            v = ((wb_ref[0:1, sl] & a0) | (wb_ref[1:2, sl] & a1)
                 | (wb_ref[2:3, sl] & a2) | (wb_ref[3:4, sl] & a3))
            smc = jnp.where(v != 0, ws_ref[0:1, sl], -1.0)
            sm_ref[0:1, sl] = smc
            vmax = jnp.maximum(vmax, jnp.max(smc))
        ok = vmax >= _SCORE_THR

        @pl.when(ok)
        def _():
            istar = _BIG
            for c in range(nchunk):
                sl = pl.ds(c * _CL, _CL)
                ids = jax.lax.broadcasted_iota(jnp.int32, (1, _CL), 1) + c * _CL
                smc = sm_ref[0:1, sl]
                istar = jnp.minimum(
                    istar, jnp.min(jnp.where(smc == vmax, ids, _BIG)))
            blk = pl.multiple_of((istar // 128) * 128, 128)
            loff = istar - blk
            lids = jax.lax.broadcasted_iota(jnp.int32, (1, 128), 1)
            hitl = lids == loff
            ioblk = iout_ref[:, pl.ds(blk, 128)]
            colv = jnp.max(jnp.where(hitl, ioblk, -1.0), axis=1, keepdims=True)
            msval = jnp.sum(jnp.where(hitl, ms_ref[0:1, pl.ds(blk, 128)], 0.0))
            availa = gtma_ref[:, :] <= 0
            candrow = jnp.where(availa, colv, -1.0)
            vrow = jnp.max(candrow)
            gida = jax.lax.broadcasted_iota(jnp.int32, (_N, 1), 0)
            mstar = jnp.min(jnp.where(candrow == vrow, gida, _BIG))
            dtm_ref[0:1, pl.ds(blk, 128)] = jnp.where(
                hitl, mstar, dtm_ref[0:1, pl.ds(blk, 128)])
            ws_ref[0:1, pl.ds(blk, 128)] = jnp.where(
                hitl, -1.0, ws_ref[0:1, pl.ds(blk, 128)])
            hita = gida == mstar
            gtma_ref[:, :] = jnp.where(hita, istar, gtma_ref[:, :])
            gtms_ref[0, mstar] = istar

            @pl.when(istar > 0)
            def _():
                wq = mstar // 32
                bq = mstar % 32
                aw_ref[0, wq] = aw_ref[0, wq] & jnp.invert(
                    jnp.left_shift(jnp.int32(1), bq))
            biou_ref[:, :] = jnp.where(hita, vrow, biou_ref[:, :])
            os_ref[:, :] = jnp.where(hita, vmax, os_ref[:, :])
            fl_ref[:, :] = jnp.where(hita, msval, fl_ref[:, :])
        return 0

    jax.lax.fori_loop(0, _N + 1, event_body, 0)

    copies = []
    for j in range(_N):
        src = jnp.maximum(gtms_ref[0, j], 0)
        cp = pltpu.make_async_copy(
            pm_ref.at[pl.ds(src, 1), :], gm_ref.at[pl.ds(j, 1), :], sems.at[j])
        cp.start()
        copies.append(cp)
    for cp in copies:
        cp.wait()
    for j in range(_N):
        matched = (gtms_ref[0, j] > -1).astype(jnp.float32)
        gm_ref[pl.ds(j, 1), :] = gm_ref[pl.ds(j, 1), :] * matched


@jax.jit
def kernel(pred_boxes, pred_scores, mask_scores, pred_masks, gt_boxes):
    pm2 = pred_masks.reshape(_M, _HW)
    pad = _MP - _M
    pbt = jnp.pad(pred_boxes, ((0, pad), (0, 0))).T
    ps2 = jnp.pad(pred_scores, (0, pad), constant_values=-1.0).reshape(1, _MP)
    ms2 = jnp.pad(mask_scores, (0, pad)).reshape(1, _MP)
    out_shape = (
        jax.ShapeDtypeStruct((_N, _HW), jnp.float32),
        jax.ShapeDtypeStruct((_N, 1), jnp.float32),
        jax.ShapeDtypeStruct((_N, 1), jnp.float32),
        jax.ShapeDtypeStruct((1, _MP), jnp.int32),
        jax.ShapeDtypeStruct((_N, 1), jnp.float32),
    )
    vmem = pl.BlockSpec(memory_space=pltpu.MemorySpace.VMEM)
    gm, fl, osc, dtm, biou = pl.pallas_call(
        _boxseg_kernel,
        out_shape=out_shape,
        in_specs=[vmem, vmem, vmem, vmem,
                  pl.BlockSpec(memory_space=pltpu.MemorySpace.HBM)],
        out_specs=(vmem, vmem, vmem, vmem, vmem),
        scratch_shapes=[
            pltpu.VMEM((_N, _MP), jnp.float32),
            pltpu.VMEM((1, _MP), jnp.float32),
            pltpu.VMEM((1, _MP), jnp.float32),
            pltpu.VMEM((_N, 1), jnp.int32),
            pltpu.SMEM((1, _N), jnp.int32),
            pltpu.VMEM((4, _MP), jnp.int32),
            pltpu.SMEM((1, 4), jnp.int32),
            pltpu.SemaphoreType.DMA((_N,)),
        ],
    )(pbt, gt_boxes, ps2, ms2, pm2)
    return (gm.reshape(_N, 28, 28), fl.reshape(_N), osc.reshape(_N),
            dtm[0, :_M], biou.reshape(_N))


# speedup vs baseline: 626.1853x; 1.0125x over previous
import jax
import jax.numpy as jnp
from jax.experimental import pallas as pl
from jax.experimental.pallas import tpu as pltpu

_IOU_THR = 0.5
_SCORE_THR = 0.05
_M = 20000
_MP = 20480
_N = 100
_HW = 784
_CL = 1024
_BIG = 2 ** 30


def _boxseg_kernel(pbt_ref, gb_ref, ps_ref, ms_ref, pm_ref,
                   gm_ref, fl_ref, os_ref, dtm_ref, biou_ref,
                   iout_ref, ws_ref, gtma_ref, gtms_ref,
                   wb_ref, aw_ref, sems):
    nchunk = _MP // _CL

    dtm_ref[:, :] = jnp.zeros((1, _MP), jnp.int32)
    fl_ref[:, :] = jnp.zeros((_N, 1), jnp.float32)
    os_ref[:, :] = jnp.zeros((_N, 1), jnp.float32)
    biou_ref[:, :] = jnp.zeros((_N, 1), jnp.float32)
    gtma_ref[:, :] = jnp.full((_N, 1), -1, jnp.int32)
    for j in range(_N):
        gtms_ref[0, j] = jnp.int32(-1)
    ps = ps_ref[:, :]
    ws_ref[:, :] = jnp.where(ps >= _SCORE_THR, ps, -1.0)

    gx1 = gb_ref[:, 0:1]
    gy1 = gb_ref[:, 1:2]
    gx2 = gb_ref[:, 2:3]
    gy2 = gb_ref[:, 3:4]
    garea = (gx2 - gx1) * (gy2 - gy1)
    for c in range(nchunk):
        sl = pl.ds(c * _CL, _CL)
        px1 = pbt_ref[0:1, sl]
        py1 = pbt_ref[1:2, sl]
        px2 = pbt_ref[2:3, sl]
        py2 = pbt_ref[3:4, sl]
        parea = (px2 - px1) * (py2 - py1)
        w = jnp.maximum(jnp.minimum(px2, gx2) - jnp.maximum(px1, gx1), 0.0)
        h = jnp.maximum(jnp.minimum(py2, gy2) - jnp.maximum(py1, gy1), 0.0)
        inter = w * h
        union = jnp.maximum(parea + garea - inter, 1e-6)
        iout_ref[:, sl] = inter / union

    gida0 = jax.lax.broadcasted_iota(jnp.int32, (_N, 1), 0)
    wts = jnp.left_shift(1, gida0 % 32)
    for c in range(nchunk):
        sl = pl.ds(c * _CL, _CL)
        wvals = jnp.where(iout_ref[:, sl] >= _IOU_THR, wts, 0)
        wb_ref[0:1, sl] = jnp.sum(wvals[0:32, :], axis=0, keepdims=True)
        wb_ref[1:2, sl] = jnp.sum(wvals[32:64, :], axis=0, keepdims=True)
        wb_ref[2:3, sl] = jnp.sum(wvals[64:96, :], axis=0, keepdims=True)
        wb_ref[3:4, sl] = jnp.sum(wvals[96:100, :], axis=0, keepdims=True)
    aw_ref[0, 0] = jnp.int32(-1)
    aw_ref[0, 1] = jnp.int32(-1)
    aw_ref[0, 2] = jnp.int32(-1)
    aw_ref[0, 3] = jnp.int32(15)

    def event_body(t, carry):
        a0 = aw_ref[0, 0]
        a1 = aw_ref[0, 1]
        a2 = aw_ref[0, 2]
        a3 = aw_ref[0, 3]
        v = ((wb_ref[0:1, :] & a0) | (wb_ref[1:2, :] & a1)
             | (wb_ref[2:3, :] & a2) | (wb_ref[3:4, :] & a3))
        sm = jnp.where(v != 0, ws_ref[0:1, :], -1.0)
        vmax = jnp.max(sm)
        ok = vmax >= _SCORE_THR

        @pl.when(ok)
        def _():
            ids = jax.lax.broadcasted_iota(jnp.int32, (1, _MP), 1)
            istar = jnp.min(jnp.where(sm == vmax, ids, _BIG))
            blk = pl.multiple_of((istar // 128) * 128, 128)
            loff = istar - blk
            lids = jax.lax.broadcasted_iota(jnp.int32, (1, 128), 1)
            hitl = lids == loff
            ioblk = iout_ref[:, pl.ds(blk, 128)]
            colv = jnp.max(jnp.where(hitl, ioblk, -1.0), axis=1, keepdims=True)
            msval = jnp.sum(jnp.where(hitl, ms_ref[0:1, pl.ds(blk, 128)], 0.0))
            availa = gtma_ref[:, :] <= 0
            candrow = jnp.where(availa, colv, -1.0)
            vrow = jnp.max(candrow)
            gida = jax.lax.broadcasted_iota(jnp.int32, (_N, 1), 0)
            mstar = jnp.min(jnp.where(candrow == vrow, gida, _BIG))
            dtm_ref[0:1, pl.ds(blk, 128)] = jnp.where(
                hitl, mstar, dtm_ref[0:1, pl.ds(blk, 128)])
            ws_ref[0:1, pl.ds(blk, 128)] = jnp.where(
                hitl, -1.0, ws_ref[0:1, pl.ds(blk, 128)])
            hita = gida == mstar
            gtma_ref[:, :] = jnp.where(hita, istar, gtma_ref[:, :])
            gtms_ref[0, mstar] = istar

            @pl.when(istar > 0)
            def _():
                wq = mstar // 32
                bq = mstar % 32
                aw_ref[0, wq] = aw_ref[0, wq] & jnp.invert(
                    jnp.left_shift(jnp.int32(1), bq))
            biou_ref[:, :] = jnp.where(hita, vrow, biou_ref[:, :])
            os_ref[:, :] = jnp.where(hita, vmax, os_ref[:, :])
            fl_ref[:, :] = jnp.where(hita, msval, fl_ref[:, :])
        return 0

    jax.lax.fori_loop(0, _N + 1, event_body, 0)

    copies = []
    for j in range(_N):
        src = jnp.maximum(gtms_ref[0, j], 0)
        cp = pltpu.make_async_copy(
            pm_ref.at[pl.ds(src, 1), :], gm_ref.at[pl.ds(j, 1), :], sems.at[j])
        cp.start()
        copies.append(cp)
    for cp in copies:
        cp.wait()
    for j in range(_N):
        matched = (gtms_ref[0, j] > -1).astype(jnp.float32)
        gm_ref[pl.ds(j, 1), :] = gm_ref[pl.ds(j, 1), :] * matched


@jax.jit
def kernel(pred_boxes, pred_scores, mask_scores, pred_masks, gt_boxes):
    pm2 = pred_masks.reshape(_M, _HW)
    pad = _MP - _M
    pbt = jnp.pad(pred_boxes, ((0, pad), (0, 0))).T
    ps2 = jnp.pad(pred_scores, (0, pad), constant_values=-1.0).reshape(1, _MP)
    ms2 = jnp.pad(mask_scores, (0, pad)).reshape(1, _MP)
    out_shape = (
        jax.ShapeDtypeStruct((_N, _HW), jnp.float32),
        jax.ShapeDtypeStruct((_N, 1), jnp.float32),
        jax.ShapeDtypeStruct((_N, 1), jnp.float32),
        jax.ShapeDtypeStruct((1, _MP), jnp.int32),
        jax.ShapeDtypeStruct((_N, 1), jnp.float32),
    )
    vmem = pl.BlockSpec(memory_space=pltpu.MemorySpace.VMEM)
    gm, fl, osc, dtm, biou = pl.pallas_call(
        _boxseg_kernel,
        out_shape=out_shape,
        in_specs=[vmem, vmem, vmem, vmem,
                  pl.BlockSpec(memory_space=pltpu.MemorySpace.HBM)],
        out_specs=(vmem, vmem, vmem, vmem, vmem),
        scratch_shapes=[
            pltpu.VMEM((_N, _MP), jnp.float32),
            pltpu.VMEM((1, _MP), jnp.float32),
            pltpu.VMEM((_N, 1), jnp.int32),
            pltpu.SMEM((1, _N), jnp.int32),
            pltpu.VMEM((4, _MP), jnp.int32),
            pltpu.SMEM((1, 4), jnp.int32),
            pltpu.SemaphoreType.DMA((_N,)),
        ],
    )(pbt, gt_boxes, ps2, ms2, pm2)
    return (gm.reshape(_N, 28, 28), fl.reshape(_N), osc.reshape(_N),
            dtm[0, :_M], biou.reshape(_N))
